# baseline (speedup 1.0000x reference)
import functools

import jax
import jax.numpy as jnp
from jax import lax
from jax.experimental import pallas as pl
from jax.experimental.pallas import tpu as pltpu
from jax.experimental.pallas import tpu_sc as plsc

N_NODES = 10000
N_EDGES = 320000
D = 128

NUM_CORES = 2
NUM_SUBCORES = 16
NUM_WORKERS = NUM_CORES * NUM_SUBCORES

CHUNK = 96
G = 9
NGROUPS = 12
EDGES_PER_TILE = NGROUPS * G * CHUNK
E_PAD = EDGES_PER_TILE * NUM_WORKERS
GROUPS_TOTAL = NUM_WORKERS * NGROUPS

N_PAD = 10240
ROWS_PER_TILE = N_PAD // NUM_SUBCORES


def _seg_sum_body(h_hbm, idx_hbm, out_hbm,
                  islab0, islab1, rows0, rows1, rows2, agg_sh,
                  semi0, semi1, semg0, semg1, semg2, sems0, sems1, sems2):
  c = lax.axis_index("c")
  s = lax.axis_index("s")
  islab = (islab0, islab1)
  rows = (rows0, rows1, rows2)
  semi = (semi0, semi1)
  semg = (semg0, semg1, semg2)
  sems = (sems0, sems1, sems2)

  row0 = s * ROWS_PER_TILE
  zrow = jnp.zeros((16,), jnp.float32)

  def zbody(r, _):
    for kk in range(D // 16):
      rows0[r, pl.ds(kk * 16, 16)] = zrow
    return 0

  lax.fori_loop(0, CHUNK, zbody, 0)
  nfull = ROWS_PER_TILE // CHUNK
  for t in range(nfull):
    pltpu.sync_copy(rows0, agg_sh.at[pl.ds(row0 + t * CHUNK, CHUNK)])
  rem = ROWS_PER_TILE - nfull * CHUNK
  pltpu.sync_copy(rows0.at[pl.ds(0, rem)],
                  agg_sh.at[pl.ds(row0 + nfull * CHUNK, rem)])
  plsc.subcore_barrier()

  wid = s * NUM_CORES + c
  gbase = wid * NGROUPS

  def fire_idx(g, p):
    pltpu.async_copy(idx_hbm.at[gbase + g], islab[p], semi[p])

  def wait_idx(p):
    pltpu.make_async_copy(idx_hbm.at[gbase], islab[p], semi[p]).wait()

  def fire_gather(isl, k, p):
    pltpu.async_copy(h_hbm.at[isl.at[k, 0]], rows[p], semg[p])

  def wait_gather(p):
    pltpu.make_async_copy(h_hbm.at[islab0.at[0, 0]], rows[p], semg[p]).wait()

  def fire_scatter(isl, k, p):
    pltpu.async_copy(rows[p], agg_sh.at[isl.at[k, 1]], sems[p], add=True)

  def wait_scatter(p):
    pltpu.make_async_copy(rows[p], agg_sh.at[islab0.at[0, 1]], sems[p]).wait()

  fire_idx(0, 0)
  fire_idx(1, 1)
  wait_idx(0)
  fire_gather(islab0, 0, 0)
  fire_gather(islab0, 1, 1)

  def one_group(g, gp):
    isl = islab[gp]

    for k in range(G):
      p = k % 3
      b2 = (k + 2) % 3
      if k == 0:
        @pl.when(g > 0)
        def _():
          wait_scatter(b2)
      else:
        wait_scatter(b2)

      if k < G - 2:
        fire_gather(isl, k + 2, b2)
      else:
        @pl.when(g + 1 < NGROUPS)
        def _():
          if k == G - 2:
            wait_idx(1 - gp)
          fire_gather(islab[1 - gp], k + 2 - G, b2)

      wait_gather(p)
      fire_scatter(isl, k, p)

    @pl.when(g + 2 < NGROUPS)
    def _():
      fire_idx(g + 2, gp)

  def body(i, _):
    one_group(2 * i, 0)
    one_group(2 * i + 1, 1)
    return 0

  lax.fori_loop(0, NGROUPS // 2, body, 0)
  wait_scatter((NGROUPS * G - 1) % 3)

  plsc.subcore_barrier()
  pltpu.sync_copy(agg_sh.at[pl.ds(row0, ROWS_PER_TILE)],
                  out_hbm.at[c, pl.ds(row0, ROWS_PER_TILE)])


def _seg_sum(h, idx):
  mesh = plsc.VectorSubcoreMesh(core_axis_name="c", subcore_axis_name="s")
  fn = pl.kernel(
      _seg_sum_body,
      out_type=jax.ShapeDtypeStruct((NUM_CORES, N_PAD, D), jnp.float32),
      mesh=mesh,
      scratch_types=[
          pltpu.VMEM((G, 2, CHUNK), jnp.int32),
          pltpu.VMEM((G, 2, CHUNK), jnp.int32),
          pltpu.VMEM((CHUNK, D), jnp.float32),
          pltpu.VMEM((CHUNK, D), jnp.float32),
          pltpu.VMEM((CHUNK, D), jnp.float32),
          pltpu.VMEM_SHARED((N_PAD, D), jnp.float32),
          pltpu.SemaphoreType.DMA,
          pltpu.SemaphoreType.DMA,
          pltpu.SemaphoreType.DMA,
          pltpu.SemaphoreType.DMA,
          pltpu.SemaphoreType.DMA,
          pltpu.SemaphoreType.DMA,
          pltpu.SemaphoreType.DMA,
          pltpu.SemaphoreType.DMA,
      ],
  )
  return fn(h, idx)


def _linear_block(h_ref, w_ref, bias_ref, o_ref):
  y = lax.dot_general(h_ref[...], w_ref[...], (((1,), (1,)), ((), ())),
                      preferred_element_type=jnp.float32,
                      precision=lax.Precision.HIGHEST)
  o_ref[...] = y + bias_ref[...]


def _gin_block(h_ref, a_ref, b_ref, w_ref, bias_ref, o_ref, *, relu):
  hh = h_ref[...] + a_ref[...] + b_ref[...]
  y = lax.dot_general(hh, w_ref[...], (((1,), (1,)), ((), ())),
                      preferred_element_type=jnp.float32,
                      precision=lax.Precision.HIGHEST)
  y = y + bias_ref[...]
  if relu:
    y = jnp.maximum(y, 0.0)
  o_ref[...] = y


_BM = 512
_GRID = N_PAD // _BM


def _linear(h, w, bias):
  return pl.pallas_call(
      _linear_block,
      out_shape=jax.ShapeDtypeStruct((N_PAD, D), jnp.float32),
      grid=(_GRID,),
      in_specs=[
          pl.BlockSpec((_BM, D), lambda i: (i, 0)),
          pl.BlockSpec((D, D), lambda i: (0, 0)),
          pl.BlockSpec((1, D), lambda i: (0, 0)),
      ],
      out_specs=pl.BlockSpec((_BM, D), lambda i: (i, 0)),
  )(h, w, bias)


def _gin_update(h, agg2, w, bias, relu):
  return pl.pallas_call(
      functools.partial(_gin_block, relu=relu),
      out_shape=jax.ShapeDtypeStruct((N_PAD, D), jnp.float32),
      grid=(_GRID,),
      in_specs=[
          pl.BlockSpec((_BM, D), lambda i: (i, 0)),
          pl.BlockSpec((_BM, D), lambda i: (i, 0)),
          pl.BlockSpec((_BM, D), lambda i: (i, 0)),
          pl.BlockSpec((D, D), lambda i: (0, 0)),
          pl.BlockSpec((1, D), lambda i: (0, 0)),
      ],
      out_specs=pl.BlockSpec((_BM, D), lambda i: (i, 0)),
  )(h, agg2[0], agg2[1], w, bias)


def kernel(x, edge_index, W_pre, b_pre, W1, b1, W2, b2):
  src = edge_index[0]
  dst = edge_index[1]
  pad = E_PAD - N_EDGES
  src = jnp.concatenate([src, jnp.zeros((pad,), jnp.int32)])
  dst = jnp.concatenate([dst, jnp.full((pad,), N_NODES, jnp.int32)])
  idx = jnp.stack([src.reshape(E_PAD // CHUNK, CHUNK),
                   dst.reshape(E_PAD // CHUNK, CHUNK)], axis=1)
  idx = idx.reshape(GROUPS_TOTAL, G, 2, CHUNK)

  xp = jnp.zeros((N_PAD, D), x.dtype).at[:N_NODES].set(x)
  bias_pre = b_pre.reshape(1, D)
  bias1 = b1.reshape(1, D)
  bias2 = b2.reshape(1, D)

  h0 = _linear(xp, W_pre, bias_pre)
  agg1 = _seg_sum(h0, idx)
  h1 = _gin_update(h0, agg1, W1, bias1, relu=True)
  agg2 = _seg_sum(h1, idx)
  out = _gin_update(h1, agg2, W2, bias2, relu=False)
  return out[:N_NODES]


# speedup vs baseline: 1.5639x; 1.5639x over previous
import functools

import jax
import jax.numpy as jnp
from jax import lax
from jax.experimental import pallas as pl
from jax.experimental.pallas import tpu as pltpu
from jax.experimental.pallas import tpu_sc as plsc

N_NODES = 10000
N_EDGES = 320000
D = 128

NUM_CORES = 2
NUM_SUBCORES = 16
NUM_WORKERS = NUM_CORES * NUM_SUBCORES

CHUNK = 128
G = 8

GROUPS_C0 = 18
GROUPS_C1 = 2
GROUPS_TOTAL = NUM_SUBCORES * (GROUPS_C0 + GROUPS_C1)
E_PAD = GROUPS_TOTAL * G * CHUNK

N_PAD = 10240
ROWS_PER_TILE = N_PAD // NUM_SUBCORES


def _seg_sum_body(h_hbm, idx_hbm, out_hbm,
                  islab0, islab1, rows0, rows1, agg_sh,
                  semi0, semi1, semg0, semg1, sems0, sems1):
  c = lax.axis_index("c")
  s = lax.axis_index("s")
  islab = (islab0, islab1)
  rows = (rows0, rows1)
  semi = (semi0, semi1)
  semg = (semg0, semg1)
  sems = (sems0, sems1)

  row0 = s * ROWS_PER_TILE
  zrow = jnp.zeros((16,), jnp.float32)

  def zbody(r, _):
    for kk in range(D // 16):
      rows0[r, pl.ds(kk * 16, 16)] = zrow
    return 0

  lax.fori_loop(0, CHUNK, zbody, 0)
  for t in range(ROWS_PER_TILE // CHUNK):
    pltpu.sync_copy(rows0, agg_sh.at[pl.ds(row0 + t * CHUNK, CHUNK)])
  plsc.subcore_barrier()

  gbase = jnp.where(c == 0, s * GROUPS_C0,
                    NUM_SUBCORES * GROUPS_C0 + s * GROUPS_C1)
  ngroups = jnp.where(c == 0, GROUPS_C0, GROUPS_C1)

  def fire_idx(g, p):
    pltpu.async_copy(idx_hbm.at[gbase + g], islab[p], semi[p])

  def wait_idx(p):
    pltpu.make_async_copy(idx_hbm.at[gbase], islab[p], semi[p]).wait()

  def fire_gather(isl, k, p):
    pltpu.async_copy(h_hbm.at[isl.at[k, 0]], rows[p], semg[p])

  def wait_gather(p):
    pltpu.make_async_copy(h_hbm.at[islab0.at[0, 0]], rows[p], semg[p]).wait()

  def fire_scatter(isl, k, p):
    pltpu.async_copy(rows[p], agg_sh.at[isl.at[k, 1]], sems[p], add=True)

  def wait_scatter(p):
    pltpu.make_async_copy(rows[p], agg_sh.at[islab0.at[0, 1]], sems[p]).wait()

  fire_idx(0, 0)
  fire_idx(1, 1)
  wait_idx(0)
  fire_gather(islab0, 0, 0)
  fire_gather(islab0, 1, 1)

  def one_group(g, gp):
    isl = islab[gp]

    for k in range(G):
      p = k % 2
      wait_gather(p)
      fire_scatter(isl, k, p)
      wait_scatter(p)
      if k < G - 2:
        fire_gather(isl, k + 2, p)
      else:
        @pl.when(g + 1 < ngroups)
        def _():
          if k == G - 2:
            wait_idx(1 - gp)
          fire_gather(islab[1 - gp], k + 2 - G, p)

    @pl.when(g + 2 < ngroups)
    def _():
      fire_idx(g + 2, gp)

  def body(i, _):
    one_group(2 * i, 0)
    one_group(2 * i + 1, 1)
    return 0

  lax.fori_loop(0, ngroups // 2, body, 0)

  plsc.subcore_barrier()
  pltpu.sync_copy(agg_sh.at[pl.ds(row0, ROWS_PER_TILE)],
                  out_hbm.at[c, pl.ds(row0, ROWS_PER_TILE)])


def _seg_sum(h, idx):
  mesh = plsc.VectorSubcoreMesh(core_axis_name="c", subcore_axis_name="s")
  fn = pl.kernel(
      _seg_sum_body,
      out_type=jax.ShapeDtypeStruct((NUM_CORES, N_PAD, D), jnp.float32),
      mesh=mesh,
      scratch_types=[
          pltpu.VMEM((G, 2, CHUNK), jnp.int32),
          pltpu.VMEM((G, 2, CHUNK), jnp.int32),
          pltpu.VMEM((CHUNK, D), jnp.float32),
          pltpu.VMEM((CHUNK, D), jnp.float32),
          pltpu.VMEM_SHARED((N_PAD, D), jnp.float32),
          pltpu.SemaphoreType.DMA,
          pltpu.SemaphoreType.DMA,
          pltpu.SemaphoreType.DMA,
          pltpu.SemaphoreType.DMA,
          pltpu.SemaphoreType.DMA,
          pltpu.SemaphoreType.DMA,
      ],
  )
  return fn(h, idx)


def _linear_block(h_ref, w_ref, bias_ref, o_ref):
  y = lax.dot_general(h_ref[...], w_ref[...], (((1,), (1,)), ((), ())),
                      preferred_element_type=jnp.float32,
                      precision=lax.Precision.HIGHEST)
  o_ref[...] = y + bias_ref[...]


def _gin_block(h_ref, a_ref, b_ref, w_ref, bias_ref, o_ref, *, relu):
  hh = h_ref[...] + a_ref[...] + b_ref[...]
  y = lax.dot_general(hh, w_ref[...], (((1,), (1,)), ((), ())),
                      preferred_element_type=jnp.float32,
                      precision=lax.Precision.HIGHEST)
  y = y + bias_ref[...]
  if relu:
    y = jnp.maximum(y, 0.0)
  o_ref[...] = y


_BM = 512
_GRID = N_PAD // _BM


def _linear(h, w, bias):
  return pl.pallas_call(
      _linear_block,
      out_shape=jax.ShapeDtypeStruct((N_PAD, D), jnp.float32),
      grid=(_GRID,),
      in_specs=[
          pl.BlockSpec((_BM, D), lambda i: (i, 0)),
          pl.BlockSpec((D, D), lambda i: (0, 0)),
          pl.BlockSpec((1, D), lambda i: (0, 0)),
      ],
      out_specs=pl.BlockSpec((_BM, D), lambda i: (i, 0)),
  )(h, w, bias)


def _gin_update(h, agg2, w, bias, relu):
  return pl.pallas_call(
      functools.partial(_gin_block, relu=relu),
      out_shape=jax.ShapeDtypeStruct((N_PAD, D), jnp.float32),
      grid=(_GRID,),
      in_specs=[
          pl.BlockSpec((_BM, D), lambda i: (i, 0)),
          pl.BlockSpec((_BM, D), lambda i: (i, 0)),
          pl.BlockSpec((_BM, D), lambda i: (i, 0)),
          pl.BlockSpec((D, D), lambda i: (0, 0)),
          pl.BlockSpec((1, D), lambda i: (0, 0)),
      ],
      out_specs=pl.BlockSpec((_BM, D), lambda i: (i, 0)),
  )(h, agg2[0], agg2[1], w, bias)


def kernel(x, edge_index, W_pre, b_pre, W1, b1, W2, b2):
  src = edge_index[0]
  dst = edge_index[1]
  pad = E_PAD - N_EDGES
  src = jnp.concatenate([src, jnp.zeros((pad,), jnp.int32)])
  dst = jnp.concatenate([dst, jnp.full((pad,), N_NODES, jnp.int32)])
  idx = jnp.stack([src.reshape(E_PAD // CHUNK, CHUNK),
                   dst.reshape(E_PAD // CHUNK, CHUNK)], axis=1)
  idx = idx.reshape(GROUPS_TOTAL, G, 2, CHUNK)

  xp = jnp.zeros((N_PAD, D), x.dtype).at[:N_NODES].set(x)
  bias_pre = b_pre.reshape(1, D)
  bias1 = b1.reshape(1, D)
  bias2 = b2.reshape(1, D)

  h0 = _linear(xp, W_pre, bias_pre)
  agg1 = _seg_sum(h0, idx)
  h1 = _gin_update(h0, agg1, W1, bias1, relu=True)
  agg2 = _seg_sum(h1, idx)
  out = _gin_update(h1, agg2, W2, bias2, relu=False)
  return out[:N_NODES]


# speedup vs baseline: 1.5920x; 1.0179x over previous
import functools

import jax
import jax.numpy as jnp
from jax import lax
from jax.experimental import pallas as pl
from jax.experimental.pallas import tpu as pltpu
from jax.experimental.pallas import tpu_sc as plsc

N_NODES = 10000
N_EDGES = 320000
D = 128

NUM_CORES = 2
NUM_SUBCORES = 16
NUM_WORKERS = NUM_CORES * NUM_SUBCORES

CHUNK = 128
G = 8

GROUPS_C0 = 18
GROUPS_C1 = 2
GROUPS_TOTAL = NUM_SUBCORES * (GROUPS_C0 + GROUPS_C1)
E_PAD = GROUPS_TOTAL * G * CHUNK

N_PAD = 10240
ROWS_PER_TILE = N_PAD // NUM_SUBCORES


def _seg_sum_body(h_hbm, idx_hbm, out_hbm,
                  islab0, islab1, rows0, rows1, agg_sh,
                  semi0, semi1, semg0, semg1, sems0, sems1):
  c = lax.axis_index("c")
  s = lax.axis_index("s")
  islab = (islab0, islab1)
  rows = (rows0, rows1)
  semi = (semi0, semi1)
  semg = (semg0, semg1)
  sems = (sems0, sems1)

  gbase = jnp.where(c == 0, s * GROUPS_C0,
                    NUM_SUBCORES * GROUPS_C0 + s * GROUPS_C1)
  ngroups = jnp.where(c == 0, GROUPS_C0, GROUPS_C1)

  pltpu.async_copy(idx_hbm.at[gbase], islab0, semi0)
  pltpu.async_copy(idx_hbm.at[gbase + 1], islab1, semi1)

  row0 = s * ROWS_PER_TILE
  zrow = jnp.zeros((16,), jnp.float32)

  def zbody(r, _):
    for kk in range(D // 16):
      rows0[r, pl.ds(kk * 16, 16)] = zrow
    return 0

  lax.fori_loop(0, CHUNK, zbody, 0)
  for t in range(ROWS_PER_TILE // CHUNK):
    pltpu.sync_copy(rows0, agg_sh.at[pl.ds(row0 + t * CHUNK, CHUNK)])
  plsc.subcore_barrier()

  def fire_idx(g, p):
    pltpu.async_copy(idx_hbm.at[gbase + g], islab[p], semi[p])

  def wait_idx(p):
    pltpu.make_async_copy(idx_hbm.at[gbase], islab[p], semi[p]).wait()

  def fire_gather(isl, k, p):
    pltpu.async_copy(h_hbm.at[isl.at[k, 0]], rows[p], semg[p])

  def wait_gather(p):
    pltpu.make_async_copy(h_hbm.at[islab0.at[0, 0]], rows[p], semg[p]).wait()

  def fire_scatter(isl, k, p):
    pltpu.async_copy(rows[p], agg_sh.at[isl.at[k, 1]], sems[p], add=True)

  def wait_scatter(p):
    pltpu.make_async_copy(rows[p], agg_sh.at[islab0.at[0, 1]], sems[p]).wait()

  wait_idx(0)
  fire_gather(islab0, 0, 0)
  fire_gather(islab0, 1, 1)

  def one_group(g, gp):
    isl = islab[gp]

    for k in range(G):
      p = k % 2
      wait_gather(p)
      fire_scatter(isl, k, p)
      wait_scatter(p)
      if k < G - 2:
        fire_gather(isl, k + 2, p)
      else:
        @pl.when(g + 1 < ngroups)
        def _():
          if k == G - 2:
            wait_idx(1 - gp)
          fire_gather(islab[1 - gp], k + 2 - G, p)

    @pl.when(g + 2 < ngroups)
    def _():
      fire_idx(g + 2, gp)

  def body(i, _):
    one_group(2 * i, 0)
    one_group(2 * i + 1, 1)
    return 0

  lax.fori_loop(0, ngroups // 2, body, 0)

  plsc.subcore_barrier()
  pltpu.sync_copy(agg_sh.at[pl.ds(row0, ROWS_PER_TILE)],
                  out_hbm.at[c, pl.ds(row0, ROWS_PER_TILE)])


def _seg_sum(h, idx):
  mesh = plsc.VectorSubcoreMesh(core_axis_name="c", subcore_axis_name="s")
  fn = pl.kernel(
      _seg_sum_body,
      out_type=jax.ShapeDtypeStruct((NUM_CORES, N_PAD, D), jnp.float32),
      mesh=mesh,
      scratch_types=[
          pltpu.VMEM((G, 2, CHUNK), jnp.int32),
          pltpu.VMEM((G, 2, CHUNK), jnp.int32),
          pltpu.VMEM((CHUNK, D), jnp.float32),
          pltpu.VMEM((CHUNK, D), jnp.float32),
          pltpu.VMEM_SHARED((N_PAD, D), jnp.float32),
          pltpu.SemaphoreType.DMA,
          pltpu.SemaphoreType.DMA,
          pltpu.SemaphoreType.DMA,
          pltpu.SemaphoreType.DMA,
          pltpu.SemaphoreType.DMA,
          pltpu.SemaphoreType.DMA,
      ],
  )
  return fn(h, idx)


def _pre_block(x_ref, wp_ref, bp_ref, w1_ref, o_ref):
  h = lax.dot_general(x_ref[...], wp_ref[...], (((1,), (1,)), ((), ())),
                      preferred_element_type=jnp.float32,
                      precision=lax.Precision.HIGHEST) + bp_ref[...]
  o_ref[...] = lax.dot_general(h, w1_ref[...], (((1,), (1,)), ((), ())),
                               preferred_element_type=jnp.float32,
                               precision=lax.Precision.HIGHEST)


def _mid_block(y_ref, a_ref, b_ref, b1_ref, w_ref, o_ref):
  h = jnp.maximum(y_ref[...] + a_ref[...] + b_ref[...] + b1_ref[...], 0.0)
  o_ref[...] = lax.dot_general(h, w_ref[...], (((1,), (1,)), ((), ())),
                               preferred_element_type=jnp.float32,
                               precision=lax.Precision.HIGHEST)


def _fin_block(z_ref, a_ref, b_ref, b2_ref, o_ref):
  o_ref[...] = z_ref[...] + a_ref[0] + b_ref[0] + b2_ref[...]


_BM = 512
_GRID = N_PAD // _BM


def _pre(x, wp, bp, w1):
  return pl.pallas_call(
      _pre_block,
      out_shape=jax.ShapeDtypeStruct((N_PAD, D), jnp.float32),
      grid=(_GRID,),
      in_specs=[
          pl.BlockSpec((_BM, D), lambda i: (i, 0)),
          pl.BlockSpec((D, D), lambda i: (0, 0)),
          pl.BlockSpec((1, D), lambda i: (0, 0)),
          pl.BlockSpec((D, D), lambda i: (0, 0)),
      ],
      out_specs=pl.BlockSpec((_BM, D), lambda i: (i, 0)),
  )(x, wp, bp, w1)


def _mid(y, agg2, b1, w):
  return pl.pallas_call(
      _mid_block,
      out_shape=jax.ShapeDtypeStruct((N_PAD, D), jnp.float32),
      grid=(_GRID,),
      in_specs=[
          pl.BlockSpec((_BM, D), lambda i: (i, 0)),
          pl.BlockSpec((_BM, D), lambda i: (i, 0)),
          pl.BlockSpec((_BM, D), lambda i: (i, 0)),
          pl.BlockSpec((1, D), lambda i: (0, 0)),
          pl.BlockSpec((D, D), lambda i: (0, 0)),
      ],
      out_specs=pl.BlockSpec((_BM, D), lambda i: (i, 0)),
  )(y, agg2[0], agg2[1], b1, w)


_BMF = 400
_GRIDF = N_NODES // _BMF


def _fin(z, agg2, b2):
  return pl.pallas_call(
      _fin_block,
      out_shape=jax.ShapeDtypeStruct((N_NODES, D), jnp.float32),
      grid=(_GRIDF,),
      in_specs=[
          pl.BlockSpec((_BMF, D), lambda i: (i, 0)),
          pl.BlockSpec((1, _BMF, D), lambda i: (0, i, 0)),
          pl.BlockSpec((1, _BMF, D), lambda i: (1, i, 0)),
          pl.BlockSpec((1, D), lambda i: (0, 0)),
      ],
      out_specs=pl.BlockSpec((_BMF, D), lambda i: (i, 0)),
  )(z, agg2, agg2, b2)


def kernel(x, edge_index, W_pre, b_pre, W1, b1, W2, b2):
  src = edge_index[0]
  dst = edge_index[1]
  pad = E_PAD - N_EDGES
  src = jnp.concatenate([src, jnp.zeros((pad,), jnp.int32)])
  dst = jnp.concatenate([dst, jnp.full((pad,), N_NODES, jnp.int32)])
  idx = jnp.stack([src.reshape(E_PAD // CHUNK, CHUNK),
                   dst.reshape(E_PAD // CHUNK, CHUNK)], axis=1)
  idx = idx.reshape(GROUPS_TOTAL, G, 2, CHUNK)

  xp = jnp.zeros((N_PAD, D), x.dtype).at[:N_NODES].set(x)
  bias_pre = b_pre.reshape(1, D)
  bias1 = b1.reshape(1, D)
  bias2 = b2.reshape(1, D)

  y1 = _pre(xp, W_pre, bias_pre, W1)
  a1 = _seg_sum(y1, idx)
  z = _mid(y1, a1, bias1, W2)
  a2 = _seg_sum(z, idx)
  return _fin(z, a2, bias2)
